# baseline (speedup 1.0000x reference)
import functools

import jax
import jax.numpy as jnp
import numpy as np
from jax import lax
from jax.experimental import pallas as pl
from jax.experimental.pallas import tpu as pltpu
from jax.experimental.pallas import tpu_sc as plsc

_N_BINS = 15
_DELTA = float(np.float32(1.0) / np.float32(_N_BINS))


def _bounds(i):
    lo = -1.0 if i == 0 else float(np.float32(i) * np.float32(_DELTA))
    up = float(np.float32(i + 1) * np.float32(_DELTA))
    return lo, up


def _stats_tc_kernel(labels_ref, logits_ref, conf_ref, corr_ref):
    x = logits_ref[0]
    c_dim, r_dim = x.shape
    m = jnp.max(x, axis=0, keepdims=True)
    e = jnp.exp(x - m)
    s = jnp.sum(e, axis=0, keepdims=True)
    conf = 1.0 / s

    iota = lax.broadcasted_iota(jnp.int32, (c_dim, r_dim), 0)
    big = jnp.int32(2**30)
    fidx = jnp.min(jnp.where(x == m, iota, big), axis=0, keepdims=True)
    labels = labels_ref[0]
    correct = (fidx == labels).astype(jnp.float32)

    conf_ref[...] = conf.reshape(r_dim)
    corr_ref[...] = correct.reshape(r_dim)


def _hist_sc_kernel(conf_hbm, corr_hbm, out_hbm, cbuf, rbuf, pbuf,
                    allp, ebuf, *, per_w, n_samples, temps_per_core):
    c = lax.axis_index("c")
    s = lax.axis_index("s")
    w = c * 16 + s
    base = w * per_w

    pltpu.sync_copy(conf_hbm.at[pl.ds(base, per_w)], cbuf)
    pltpu.sync_copy(corr_hbm.at[pl.ds(base, per_w)], rbuf)

    zeros16 = jnp.zeros((16,), jnp.float32)
    lane = lax.iota(jnp.int32, 16)

    def _binned(cv, rv, carry):
        accs = []
        qv = rv + 4096.0
        for j in range(_N_BINS):
            lo, up = _bounds(j)
            m = (cv > lo) & (cv <= up)
            accs.append(carry[2 * j] + jnp.where(m, cv, zeros16))
            accs.append(carry[2 * j + 1] + jnp.where(m, qv, zeros16))
        return accs

    def body(i, carry):
        c0 = _binned(cbuf[pl.ds(i * 32, 16)], rbuf[pl.ds(i * 32, 16)], carry)
        c1 = _binned(cbuf[pl.ds(i * 32 + 16, 16)],
                     rbuf[pl.ds(i * 32 + 16, 16)], c0)
        return tuple(c1)

    init = tuple(jnp.zeros((16,), jnp.float32) for _ in range(2 * _N_BINS))
    fin = lax.fori_loop(0, per_w // 32, body, init)

    for ty in range(2):
        acc = zeros16
        for j in range(_N_BINS):
            v = fin[2 * j + ty]
            sb = v[0]
            for k in range(1, 16):
                sb = sb + v[k]
            acc = acc + jnp.where(lane == j, jnp.full((16,), sb, jnp.float32),
                                  zeros16)
        if ty == 0:
            pbuf[pl.ds(0, 16)] = acc
        else:
            qi = acc.astype(jnp.int32)
            cnt = qi >> 12
            cor = qi - (cnt << 12)
            pbuf[pl.ds(16, 16)] = cor.astype(jnp.float32)
            pbuf[pl.ds(32, 16)] = cnt.astype(jnp.float32)

    pltpu.sync_copy(pbuf, allp.at[pl.ds(s * 48, 48)])
    plsc.subcore_barrier()

    @pl.when(s == 0)
    def _finish():
        pltpu.sync_copy(allp, ebuf)
        inv_n = float(1.0 / n_samples)
        w_per_t = 16 // temps_per_core
        z16 = jnp.zeros((16,), jnp.float32)
        evec = z16
        for ti in range(temps_per_core):
            cv = z16
            rv = z16
            nv = z16
            for q in range(w_per_t):
                wi = ti * w_per_t + q
                cv = cv + ebuf[pl.ds(wi * 48, 16)]
                rv = rv + ebuf[pl.ds(wi * 48 + 16, 16)]
                nv = nv + ebuf[pl.ds(wi * 48 + 32, 16)]
            contrib = jnp.where(nv > z16, jnp.abs((cv - rv) * inv_n), z16)
            ece_t = contrib[0]
            for k in range(1, 16):
                ece_t = ece_t + contrib[k]
            ece_v = jnp.full((16,), ece_t, jnp.float32)
            evec = evec + jnp.where(lane == ti, ece_v, z16)
        pbuf[pl.ds(0, 16)] = evec
        pltpu.sync_copy(pbuf.at[pl.ds(0, 16)], out_hbm.at[pl.ds(c * 16, 16)])


def kernel(logits, labels):
    T, N, C = logits.shape
    R = 2048
    while N % R != 0:
        R //= 2
    NB = N // R

    logits_t = jnp.transpose(logits, (0, 2, 1))

    conf_flat, corr_flat = pl.pallas_call(
        _stats_tc_kernel,
        grid=(T, NB),
        in_specs=[
            pl.BlockSpec((1, 1, R), lambda t, nb: (nb, 0, 0)),
            pl.BlockSpec((1, C, R), lambda t, nb: (t, 0, nb)),
        ],
        out_specs=[
            pl.BlockSpec((R,), lambda t, nb: (t * NB + nb,)),
            pl.BlockSpec((R,), lambda t, nb: (t * NB + nb,)),
        ],
        out_shape=[jax.ShapeDtypeStruct((T * N,), jnp.float32),
                   jax.ShapeDtypeStruct((T * N,), jnp.float32)],
    )(labels.reshape(NB, 1, R), logits_t)

    per_w = (T * N) // 32
    temps_per_core = T // 2
    mesh = plsc.VectorSubcoreMesh(core_axis_name="c", subcore_axis_name="s")
    hist_call = pl.kernel(
        functools.partial(_hist_sc_kernel, per_w=per_w, n_samples=N,
                          temps_per_core=temps_per_core),
        mesh=mesh,
        out_type=jax.ShapeDtypeStruct((32,), jnp.float32),
        scratch_types=[
            pltpu.VMEM((per_w,), jnp.float32),
            pltpu.VMEM((per_w,), jnp.float32),
            pltpu.VMEM((48,), jnp.float32),
            pltpu.VMEM_SHARED((768,), jnp.float32),
            pltpu.VMEM((768,), jnp.float32),
        ],
    )
    out2 = hist_call(conf_flat, corr_flat)
    return jnp.concatenate(
        [out2[:temps_per_core], out2[16:16 + temps_per_core]])


# speedup vs baseline: 1.0855x; 1.0855x over previous
import functools

import jax
import jax.numpy as jnp
import numpy as np
from jax import lax
from jax.experimental import pallas as pl
from jax.experimental.pallas import tpu as pltpu
from jax.experimental.pallas import tpu_sc as plsc

_N_BINS = 15
_DELTA = float(np.float32(1.0) / np.float32(_N_BINS))


def _bounds(i):
    lo = -1.0 if i == 0 else float(np.float32(i) * np.float32(_DELTA))
    up = float(np.float32(i + 1) * np.float32(_DELTA))
    return lo, up


def _stats_tc_kernel(labels_ref, logits_ref, conf_ref, corr_ref):
    x = logits_ref[0]
    c_dim, r_dim = x.shape
    m = jnp.max(x, axis=0, keepdims=True)
    e = jnp.exp(x - m)
    s = jnp.sum(e, axis=0, keepdims=True)
    conf = 1.0 / s

    iota = lax.broadcasted_iota(jnp.int32, (c_dim, r_dim), 0)
    big = jnp.int32(2**30)
    fidx = jnp.min(jnp.where(x == m, iota, big), axis=0, keepdims=True)
    labels = labels_ref[0]
    correct = (fidx == labels).astype(jnp.float32)

    conf_ref[...] = conf.reshape(r_dim)
    corr_ref[...] = correct.reshape(r_dim)


def _hist_sc_kernel(conf_hbm, corr_hbm, out_hbm, cbuf, rbuf, pbuf,
                    allp, ebuf, *, per_w, n_samples, temps_per_core):
    c = lax.axis_index("c")
    s = lax.axis_index("s")
    w = c * 16 + s
    base = w * per_w

    pltpu.sync_copy(conf_hbm.at[pl.ds(base, per_w)], cbuf)
    pltpu.sync_copy(corr_hbm.at[pl.ds(base, per_w)], rbuf)

    zeros16 = jnp.zeros((16,), jnp.float32)
    lane = lax.iota(jnp.int32, 16)
    ones16 = jnp.ones((16,), jnp.float32)

    def body(i, carry):
        cv = cbuf[pl.ds(i * 16, 16)]
        rv = rbuf[pl.ds(i * 16, 16)]
        accs = []
        for j in range(_N_BINS):
            lo, up = _bounds(j)
            m = (cv > lo) & (cv <= up)
            accs.append(carry[3 * j] + jnp.where(m, cv, zeros16))
            accs.append(carry[3 * j + 1] + jnp.where(m, rv, zeros16))
            accs.append(carry[3 * j + 2] + jnp.where(m, ones16, zeros16))
        return tuple(accs)

    init = tuple(jnp.zeros((16,), jnp.float32) for _ in range(3 * _N_BINS))
    fin = lax.fori_loop(0, per_w // 16, body, init)

    for ty in range(3):
        acc = zeros16
        for j in range(_N_BINS):
            v = fin[3 * j + ty]
            sb = v[0]
            for k in range(1, 16):
                sb = sb + v[k]
            acc = acc + jnp.where(lane == j, jnp.full((16,), sb, jnp.float32),
                                  zeros16)
        pbuf[pl.ds(ty * 16, 16)] = acc

    pltpu.sync_copy(pbuf, allp.at[pl.ds(s * 48, 48)])
    plsc.subcore_barrier()

    @pl.when(s == 0)
    def _finish():
        pltpu.sync_copy(allp, ebuf)
        inv_n = float(1.0 / n_samples)
        w_per_t = 16 // temps_per_core
        z16 = jnp.zeros((16,), jnp.float32)
        evec = z16
        for ti in range(temps_per_core):
            cv = z16
            rv = z16
            nv = z16
            for q in range(w_per_t):
                wi = ti * w_per_t + q
                cv = cv + ebuf[pl.ds(wi * 48, 16)]
                rv = rv + ebuf[pl.ds(wi * 48 + 16, 16)]
                nv = nv + ebuf[pl.ds(wi * 48 + 32, 16)]
            contrib = jnp.where(nv > z16, jnp.abs((cv - rv) * inv_n), z16)
            ece_t = contrib[0]
            for k in range(1, 16):
                ece_t = ece_t + contrib[k]
            ece_v = jnp.full((16,), ece_t, jnp.float32)
            evec = evec + jnp.where(lane == ti, ece_v, z16)
        pbuf[pl.ds(0, 16)] = evec
        pltpu.sync_copy(pbuf.at[pl.ds(0, 16)], out_hbm.at[pl.ds(c * 16, 16)])


def kernel(logits, labels):
    T, N, C = logits.shape
    R = 2048
    while N % R != 0:
        R //= 2
    NB = N // R

    logits_t = jnp.transpose(logits, (0, 2, 1))

    conf_flat, corr_flat = pl.pallas_call(
        _stats_tc_kernel,
        grid=(T, NB),
        in_specs=[
            pl.BlockSpec((1, 1, R), lambda t, nb: (nb, 0, 0)),
            pl.BlockSpec((1, C, R), lambda t, nb: (t, 0, nb)),
        ],
        out_specs=[
            pl.BlockSpec((R,), lambda t, nb: (t * NB + nb,)),
            pl.BlockSpec((R,), lambda t, nb: (t * NB + nb,)),
        ],
        out_shape=[jax.ShapeDtypeStruct((T * N,), jnp.float32),
                   jax.ShapeDtypeStruct((T * N,), jnp.float32)],
    )(labels.reshape(NB, 1, R), logits_t)

    per_w = (T * N) // 32
    temps_per_core = T // 2
    mesh = plsc.VectorSubcoreMesh(core_axis_name="c", subcore_axis_name="s")
    hist_call = pl.kernel(
        functools.partial(_hist_sc_kernel, per_w=per_w, n_samples=N,
                          temps_per_core=temps_per_core),
        mesh=mesh,
        out_type=jax.ShapeDtypeStruct((32,), jnp.float32),
        scratch_types=[
            pltpu.VMEM((per_w,), jnp.float32),
            pltpu.VMEM((per_w,), jnp.float32),
            pltpu.VMEM((48,), jnp.float32),
            pltpu.VMEM_SHARED((768,), jnp.float32),
            pltpu.VMEM((768,), jnp.float32),
        ],
    )
    out2 = hist_call(conf_flat, corr_flat)
    return jnp.concatenate(
        [out2[:temps_per_core], out2[16:16 + temps_per_core]])
